# baseline (speedup 1.0000x reference)
import functools

import numpy as np
import jax
import jax.numpy as jnp
from jax import lax
from jax.experimental import pallas as pl
from jax.experimental.pallas import tpu as pltpu
from jax.experimental.pallas import tpu_sc as plsc

_DIMS = (119, 5, 12, 12, 10, 6, 6, 2, 2)
_NF = 9
_SD = 16
_D = 128
_V = int(sum(_DIMS))
_OFFS = np.array([0] + list(np.cumsum(_DIMS)[:-1]), dtype=np.int32)

_NWORK = 32
_WR = 64
_CH_ROWS = 8
_CH_IDX = _CH_ROWS * _NF
_NCH = _WR // _CH_ROWS
_ROWS_PER_STEP = _NWORK * _WR

_ROWMAP = np.asarray(
    [[[sid * _WR + ch * _CH_ROWS + j // _NF for j in range(_CH_IDX)]
      for ch in range(_NCH)]
     for sid in range(16)],
    dtype=np.int32,
)


def _prep_body(x_ref, offs_ref, idx_ref):
    codes = x_ref[:, :_NF].astype(jnp.int32)
    idx_ref[...] = codes + offs_ref[0, :][None, :]


def _linear_body(e_ref, x_ref, w1_ref, w2_ref, o_ref):
    e = e_ref[...]
    s = x_ref[:, _NF:_NF + _SD]
    o_ref[...] = (
        e
        + jnp.dot(e, w1_ref[...], preferred_element_type=jnp.float32)
        + jnp.dot(s, w2_ref[...], preferred_element_type=jnp.float32)
    )


def _sc_gather_sum(idx2, rowmap, table, zeros, npad):
    rpw = npad // _NWORK
    nwin = rpw // _WR
    mesh = plsc.VectorSubcoreMesh(core_axis_name="c", subcore_axis_name="s")

    @functools.partial(
        pl.kernel,
        out_type=jax.ShapeDtypeStruct((npad, _D), jnp.float32),
        mesh=mesh,
        scratch_types=[
            pltpu.VMEM((_NCH, _CH_IDX), jnp.int32),
            pltpu.VMEM((_NCH, _CH_IDX), jnp.int32),
            pltpu.VMEM((_NCH * _CH_IDX, _D), jnp.float32),
            pltpu.VMEM_SHARED((16 * _WR, _D), jnp.float32),
            pltpu.VMEM((_WR, _D), jnp.float32),
            pltpu.SemaphoreType.DMA,
        ],
    )
    def sc_kernel(idx_hbm, map_hbm, tab_hbm, zero_hbm, out_hbm,
                  idx_v, map_v, g_v, acc_sh, zero_v, sem):
        sid = lax.axis_index("s")
        wid = sid * 2 + lax.axis_index("c")
        base = wid * rpw
        pltpu.sync_copy(map_hbm.at[sid], map_v)
        pltpu.sync_copy(zero_hbm, zero_v)

        @pl.loop(0, nwin)
        def _(win):
            r0 = pl.multiple_of(base + win * _WR, _WR)
            pltpu.sync_copy(
                idx_hbm.at[pl.ds(pl.multiple_of(r0 // _CH_ROWS, 8), _NCH)],
                idx_v)
            for ch in range(_NCH):
                pltpu.async_copy(
                    tab_hbm.at[idx_v.at[ch]],
                    g_v.at[pl.ds(ch * _CH_IDX, _CH_IDX)],
                    sem,
                )
            pltpu.sync_copy(zero_v, acc_sh.at[pl.ds(sid * _WR, _WR)])
            for ch in range(_NCH):
                pltpu.make_async_copy(
                    tab_hbm.at[idx_v.at[ch]],
                    g_v.at[pl.ds(ch * _CH_IDX, _CH_IDX)],
                    sem,
                ).wait()
            for ch in range(_NCH):
                pltpu.sync_copy(
                    g_v.at[pl.ds(ch * _CH_IDX, _CH_IDX)],
                    acc_sh.at[map_v.at[ch]],
                    add=True,
                )
            pltpu.sync_copy(acc_sh.at[pl.ds(sid * _WR, _WR)],
                            out_hbm.at[pl.ds(r0, _WR)])

    return sc_kernel(idx2, rowmap, table, zeros)


def _prep(x, n, bp=4000):
    offs = jnp.asarray(np.broadcast_to(_OFFS[None, :], (8, _NF)))
    return pl.pallas_call(
        _prep_body,
        grid=(n // bp,),
        in_specs=[
            pl.BlockSpec((bp, _NF + _SD), lambda i: (i, 0)),
            pl.BlockSpec((8, _NF), lambda i: (0, 0)),
        ],
        out_specs=pl.BlockSpec((bp, _NF), lambda i: (i, 0)),
        out_shape=jax.ShapeDtypeStruct((n, _NF), jnp.int32),
    )(x, offs)


def _linear(e, x, w1, w2, n, bt=4000):
    return pl.pallas_call(
        _linear_body,
        grid=(n // bt,),
        in_specs=[
            pl.BlockSpec((bt, _D), lambda i: (i, 0)),
            pl.BlockSpec((bt, _NF + _SD), lambda i: (i, 0)),
            pl.BlockSpec((_D, _D), lambda i: (0, 0)),
            pl.BlockSpec((_SD, _D), lambda i: (0, 0)),
        ],
        out_specs=pl.BlockSpec((bt, _D), lambda i: (i, 0)),
        out_shape=jax.ShapeDtypeStruct((n, _D), jnp.float32),
    )(e, x, w1, w2)


def kernel(x, emb0, emb1, emb2, emb3, emb4, emb5, emb6, emb7, emb8, W_scalar):
    n = x.shape[0]
    table = jnp.concatenate(
        [emb0, emb1, emb2, emb3, emb4, emb5, emb6, emb7, emb8], axis=0)
    idx = _prep(x, n)
    npad = -(-n // _ROWS_PER_STEP) * _ROWS_PER_STEP
    idx_flat = jnp.pad(idx, ((0, npad - n), (0, 0)))
    idx2 = idx_flat.reshape(npad * _NF // _CH_IDX, _CH_IDX)
    e = _sc_gather_sum(
        idx2,
        jnp.asarray(_ROWMAP),
        table,
        jnp.zeros((_WR, _D), jnp.float32),
        npad,
    )
    return _linear(e, x, W_scalar[:_D], W_scalar[_D:], n)
